# baseline (speedup 1.0000x reference)
import jax
import jax.numpy as jnp
from jax.experimental import pallas as pl

_BETA = 0.3
_TEMP = 0.07
_EPS = 1e-06

_S, _C, _H, _W = 4, 512, 64, 64
_B = 8
_P = _H * _W


def _proto_kernel(sf_ref, sm_ref, out_ref):
    i = pl.program_id(0)
    f = sf_ref[0]
    m = sm_ref[0]
    ps = jnp.sum(f, axis=1)
    ms = jnp.sum(f * m, axis=1)
    mc = jnp.sum(m)
    upd = jnp.concatenate(
        [ps[None, :], ms[None, :],
         jnp.full((1, _C), mc, jnp.float32),
         jnp.zeros((5, _C), jnp.float32)], axis=0)

    @pl.when(i == 0)
    def _init():
        out_ref[...] = jnp.zeros_like(out_ref)

    out_ref[...] += upd


def _logits_kernel(q_ref, p_ref, neg_ref, pos_ref):
    ps = p_ref[0, :]
    ms = p_ref[1, :]
    mc = p_ref[2, :]
    mean_proto = ps * (1.0 / (_S * _P))
    masked_proto = ms / jnp.maximum(mc, _EPS)
    gp_raw = jnp.where(mc < _EPS, mean_proto, masked_proto)
    gp_norm = jnp.sqrt(jnp.sum(gp_raw * gp_raw))
    gp = gp_raw / jnp.maximum(gp_norm, 1e-12)

    q = q_ref[0]
    dot = jax.lax.dot_general(
        gp[None, :], q, (((1,), (0,)), ((), ())),
        preferred_element_type=jnp.float32)[0]
    sq = jnp.sum(q * q, axis=0)
    s = _BETA * dot / jnp.maximum(jnp.sqrt(sq), 1e-12)
    pos = s * (1.0 / _TEMP)
    pos_ref[0, 0, :] = pos
    neg_ref[0, 0, :] = -pos


def kernel(support_feats, support_masks, query_feats):
    sf = support_feats.reshape(_S, _C, _P)
    sm = support_masks.reshape(_S, 1, _P)
    q = query_feats.reshape(_B, _C, _P)

    proto = pl.pallas_call(
        _proto_kernel,
        grid=(_S,),
        in_specs=[
            pl.BlockSpec((1, _C, _P), lambda i: (i, 0, 0)),
            pl.BlockSpec((1, 1, _P), lambda i: (i, 0, 0)),
        ],
        out_specs=pl.BlockSpec((8, _C), lambda i: (0, 0)),
        out_shape=jax.ShapeDtypeStruct((8, _C), jnp.float32),
    )(sf, sm)

    neg, pos = pl.pallas_call(
        _logits_kernel,
        grid=(_B,),
        in_specs=[
            pl.BlockSpec((1, _C, _P), lambda i: (i, 0, 0)),
            pl.BlockSpec((8, _C), lambda i: (0, 0)),
        ],
        out_specs=[
            pl.BlockSpec((1, 1, _P), lambda i: (i, 0, 0)),
            pl.BlockSpec((1, 1, _P), lambda i: (i, 0, 0)),
        ],
        out_shape=[
            jax.ShapeDtypeStruct((_B, 1, _P), jnp.float32),
            jax.ShapeDtypeStruct((_B, 1, _P), jnp.float32),
        ],
    )(q, proto)

    logits = jnp.concatenate((neg, pos), axis=1).reshape(_B, 2, _H, _W)
    return logits
